# baseline (speedup 1.0000x reference)
import functools

import jax
import jax.numpy as jnp
from jax import lax
from jax.experimental import pallas as pl
from jax.experimental.pallas import tpu as pltpu
from jax.experimental.pallas import tpu_sc as plsc

NC = 2
NS = 16
NW = NC * NS


def _node_table(state, eattr, nattr, A, B):
    N, D = state.shape
    D2 = D
    BN = 1000

    def pack16(x):
        f = x.astype(jnp.bfloat16).astype(jnp.float32)
        return jax.lax.bitcast_convert_type(f, jnp.int32)

    def body(s_ref, e_ref, na_ref, Am_ref, Bm_ref, wm_ref, Aa_ref, Ba_ref,
             wa_ref, P_ref):
        u = s_ref[...] @ Am_ref[...] + e_ref[...] @ Bm_ref[...] \
            + na_ref[...] * wm_ref[...]
        v = s_ref[...] @ Aa_ref[...] + e_ref[...] @ Ba_ref[...] \
            + na_ref[...] * wa_ref[...]
        uw = jnp.bitwise_and(jnp.right_shift(pack16(u), 16), jnp.int32(65535))
        vw = jnp.bitwise_and(pack16(v), jnp.int32(-65536))
        P_ref[...] = jnp.bitwise_or(uw, vw)

    wspec = [
        pl.BlockSpec((D, D2), lambda i: (0, 0)),
        pl.BlockSpec((D, D2), lambda i: (0, 0)),
        pl.BlockSpec((1, D2), lambda i: (0, 0)),
    ]
    return pl.pallas_call(
        body,
        grid=(N // BN,),
        in_specs=[
            pl.BlockSpec((BN, D), lambda i: (i, 0)),
            pl.BlockSpec((BN, D), lambda i: (i, 0)),
            pl.BlockSpec((BN, 1), lambda i: (i, 0)),
        ] + wspec + wspec,
        out_specs=pl.BlockSpec((BN, D2), lambda i: (i, 0)),
        out_shape=jax.ShapeDtypeStruct((N, D2), jnp.int32),
    )(state, eattr, nattr, *A, *B)


def _gather_diff(P, src3d, dst3d):
    N, D2 = P.shape
    _, NCH, C = src3d.shape
    E = NW * NCH * C
    NB = 4
    OFF = 2
    NCYC = (NCH + OFF + NB - 1) // NB

    mesh = plsc.VectorSubcoreMesh(core_axis_name="c", subcore_axis_name="s")

    @functools.partial(
        pl.kernel,
        out_type=(jax.ShapeDtypeStruct((E, D2), jnp.int32),
                  jax.ShapeDtypeStruct((E, D2), jnp.int32)),
        mesh=mesh,
        scratch_types=[
            pltpu.VMEM((NCH, C), jnp.int32),
            pltpu.VMEM((NCH, C), jnp.int32),
            pltpu.VMEM((NB, C, D2), jnp.int32),
            pltpu.VMEM((NB, C, D2), jnp.int32),
        ] + [pltpu.SemaphoreType.DMA] * (2 * NB),
    )
    def gather_k(P_hbm, src_hbm, dst_hbm, outs_hbm, outd_hbm, sbuf, dbuf,
                 bufA, bufB, *sems):
        semg = sems[:NB]
        semw = sems[NB:]
        wid = lax.axis_index("s") * NC + lax.axis_index("c")
        crow0 = wid * NCH

        pltpu.sync_copy(src_hbm.at[wid], sbuf)
        pltpu.sync_copy(dst_hbm.at[wid], dbuf)

        def drain_wb(b):
            pltpu.make_async_copy(bufA.at[b], outs_hbm.at[pl.ds(0, C)],
                                  semw[b]).wait()
            pltpu.make_async_copy(bufB.at[b], outd_hbm.at[pl.ds(0, C)],
                                  semw[b]).wait()

        def cycle(g, _):
            for b in range(NB):
                j = g * NB + b
                bw = (b - OFF) % NB
                jw = g * NB + b - OFF

                @pl.when(j < NCH)
                def _():
                    @pl.when(j >= NB)
                    def _():
                        drain_wb(b)
                    pltpu.async_copy(P_hbm.at[sbuf.at[j]], bufA.at[b], semg[b])
                    pltpu.async_copy(P_hbm.at[dbuf.at[j]], bufB.at[b], semg[b])

                @pl.when(jnp.logical_and(jw >= 0, jw < NCH))
                def _():
                    pltpu.make_async_copy(P_hbm.at[sbuf.at[bw]], bufA.at[bw],
                                          semg[bw]).wait()
                    pltpu.make_async_copy(P_hbm.at[dbuf.at[bw]], bufB.at[bw],
                                          semg[bw]).wait()
                    rows = pl.ds((crow0 + jw) * C, C)
                    pltpu.async_copy(bufA.at[bw], outs_hbm.at[rows], semw[bw])
                    pltpu.async_copy(bufB.at[bw], outd_hbm.at[rows], semw[bw])
            return 0

        lax.fori_loop(0, NCYC, cycle, 0)
        for b in range(NB):
            drain_wb(b)

    return gather_k(P, src3d, dst3d)


def _edge_mlp(Gs, Gd, ef, Wefm, b1m, Wefa, b1a, W2m, b2m, W2a, b2a):
    E, D = Gs.shape
    DE = ef.shape[1]
    BE = 2560
    bf = jnp.bfloat16

    def unpack(g):
        lo = jax.lax.bitcast_convert_type(jnp.left_shift(g, 16), jnp.float32)
        hi = jax.lax.bitcast_convert_type(
            jnp.bitwise_and(g, jnp.int32(-65536)), jnp.float32)
        return lo, hi

    def body(Gs_ref, Gd_ref, ef_ref, Wefm_ref, b1m_ref, Wefa_ref, b1a_ref,
             W2m_ref, b2m_ref, W2a_ref, b2a_ref, o_ref):
        sm, sa = unpack(Gs_ref[...])
        dm, da = unpack(Gd_ref[...])
        gm = sm - dm
        ga = sa - da
        efv = ef_ref[...]
        hm = jnp.maximum(gm + efv @ Wefm_ref[...] + b1m_ref[...], 0.0).astype(bf)
        ha = jnp.maximum(ga + efv @ Wefa_ref[...] + b1a_ref[...], 0.0).astype(bf)
        msg = jax.lax.dot(hm, W2m_ref[...].astype(bf),
                          preferred_element_type=jnp.float32) + b2m_ref[...]
        att = jax.lax.dot(ha, W2a_ref[...].astype(bf),
                          preferred_element_type=jnp.float32) + b2a_ref[...]
        o_ref[...] = msg * jax.nn.sigmoid(att)

    wspec = [
        pl.BlockSpec((DE, D), lambda i: (0, 0)),
        pl.BlockSpec((1, D), lambda i: (0, 0)),
    ]
    return pl.pallas_call(
        body,
        grid=(E // BE,),
        in_specs=[
            pl.BlockSpec((BE, D), lambda i: (i, 0)),
            pl.BlockSpec((BE, D), lambda i: (i, 0)),
            pl.BlockSpec((BE, DE), lambda i: (i, 0)),
        ] + wspec + wspec + [
            pl.BlockSpec((D, D), lambda i: (0, 0)),
            pl.BlockSpec((1, D), lambda i: (0, 0)),
            pl.BlockSpec((D, D), lambda i: (0, 0)),
            pl.BlockSpec((1, D), lambda i: (0, 0)),
        ],
        out_specs=pl.BlockSpec((BE, D), lambda i: (i, 0)),
        out_shape=jax.ShapeDtypeStruct((E, D), jnp.float32),
    )(Gs, Gd, ef, Wefm, b1m, Wefa, b1a, W2m, b2m, W2a, b2a)


def _scatter_add(m, dst3d, N):
    E, D = m.shape
    _, NCH, C2 = dst3d.shape
    M = 3
    NCYC = (NCH + M - 1) // M
    CZ = 16
    NZCH = N // CZ
    ZPT = (NZCH + NS - 1) // NS

    mesh = plsc.VectorSubcoreMesh(core_axis_name="c", subcore_axis_name="s")

    @functools.partial(
        pl.kernel,
        out_type=jax.ShapeDtypeStruct((NC, N, D), jnp.float32),
        mesh=mesh,
        scratch_types=[
            pltpu.VMEM_SHARED((N, D), jnp.float32),
            pltpu.VMEM((NCH, C2), jnp.int32),
            pltpu.VMEM((M, C2, D), jnp.float32),
            pltpu.VMEM((CZ, D), jnp.float32),
        ] + [pltpu.SemaphoreType.DMA] * (2 * M),
    )
    def scatter_k(m_hbm, dst_hbm, out_hbm, acc, dbuf, mbuf, zbuf, *sems):
        semL = sems[:M]
        semS = sems[M:]
        c = lax.axis_index("c")
        s = lax.axis_index("s")
        wid = s * NC + c
        crow0 = wid * NCH

        def zrow(r, _):
            for k in range(D // 16):
                zbuf[r, pl.ds(k * 16, 16)] = jnp.zeros((16,), jnp.float32)
            return 0

        lax.fori_loop(0, CZ, zrow, 0)

        def zchunk(kk, _):
            jz = kk * NS + s

            @pl.when(jz < NZCH)
            def _():
                pltpu.sync_copy(zbuf, acc.at[pl.ds(jz * CZ, CZ)])
            return 0

        lax.fori_loop(0, ZPT, zchunk, 0)
        pltpu.sync_copy(dst_hbm.at[wid], dbuf)
        plsc.subcore_barrier()

        def cycle(g, _):
            for b in range(M):
                j = g * M + b
                bs = (b - M // 2) % M
                js = j - M // 2

                @pl.when(j < NCH)
                def _():
                    @pl.when(j >= M)
                    def _():
                        pltpu.make_async_copy(
                            mbuf.at[b], acc.at[dbuf.at[0]], semS[b]).wait()
                    pltpu.async_copy(m_hbm.at[pl.ds((crow0 + j) * C2, C2)],
                                     mbuf.at[b], semL[b])

                @pl.when(jnp.logical_and(js >= 0, js < NCH))
                def _():
                    pltpu.make_async_copy(
                        m_hbm.at[pl.ds(0, C2)], mbuf.at[bs], semL[bs]).wait()
                    pltpu.async_copy(mbuf.at[bs], acc.at[dbuf.at[js]], semS[bs],
                                     add=True)
            return 0

        lax.fori_loop(0, NCYC + 1, cycle, 0)
        for b in range(M):
            pltpu.make_async_copy(mbuf.at[b], acc.at[dbuf.at[0]], semS[b]).wait()
        plsc.subcore_barrier()

        def dchunk(kk, _):
            jz = kk * NS + s

            @pl.when(jz < NZCH)
            def _():
                rows = pl.ds(jz * CZ, CZ)
                pltpu.sync_copy(acc.at[rows], zbuf)
                pltpu.sync_copy(zbuf, out_hbm.at[c, rows])
            return 0

        lax.fori_loop(0, ZPT, dchunk, 0)

    return scatter_k(m, dst3d)


def _gru(parts_a, parts_b, state, Wih, Whh, bih, bhh):
    N, D = state.shape
    D3 = Wih.shape[1]
    BN = 1000

    def body(pa_ref, pb_ref, s_ref, Wih_ref, Whh_ref, bih_ref, bhh_ref, o_ref):
        x = pa_ref[0] + pa_ref[1] + pb_ref[0] + pb_ref[1]
        h = s_ref[...]
        gi = x @ Wih_ref[...] + bih_ref[...]
        gh = h @ Whh_ref[...] + bhh_ref[...]
        r = jax.nn.sigmoid(gi[:, :D] + gh[:, :D])
        z = jax.nn.sigmoid(gi[:, D:2 * D] + gh[:, D:2 * D])
        n = jnp.tanh(gi[:, 2 * D:] + r * gh[:, 2 * D:])
        o_ref[...] = (1.0 - z) * n + z * h

    return pl.pallas_call(
        body,
        grid=(N // BN,),
        in_specs=[
            pl.BlockSpec((2, BN, D), lambda i: (0, i, 0)),
            pl.BlockSpec((2, BN, D), lambda i: (0, i, 0)),
            pl.BlockSpec((BN, D), lambda i: (i, 0)),
            pl.BlockSpec((D, D3), lambda i: (0, 0)),
            pl.BlockSpec((D, D3), lambda i: (0, 0)),
            pl.BlockSpec((1, D3), lambda i: (0, 0)),
            pl.BlockSpec((1, D3), lambda i: (0, 0)),
        ],
        out_specs=pl.BlockSpec((BN, D), lambda i: (i, 0)),
        out_shape=jax.ShapeDtypeStruct((N, D), jnp.float32),
    )(parts_a, parts_b, state, Wih, Whh, bih, bhh)


def kernel(node_feat, edge, edge_feat, node_attributes, edge_attributes,
           msg_W1, msg_b1, msg_W2, msg_b2, att_W1, att_b1, att_W2, att_b2,
           gru_Wih, gru_Whh, gru_bih, gru_bhh):
    N, D = node_feat.shape
    E = edge.shape[0]
    DE = edge_feat.shape[1]
    C = 80
    NCHT = E // (NW * C)
    split = [(0, (NCHT + 1) // 2), ((NCHT + 1) // 2, NCHT)]
    eattr = edge_attributes[0]
    nattr = node_attributes[0][:, None]

    Am, Wefm, Bm, wm = (msg_W1[:D], msg_W1[D:D + DE],
                        msg_W1[D + DE:D + DE + D], msg_W1[D + DE + D:])
    Aa, Wefa, Ba, wa = (att_W1[:D], att_W1[D:D + DE],
                        att_W1[D + DE:D + DE + D], att_W1[D + DE + D:])

    P = _node_table(node_feat, eattr, nattr, (Am, Bm, wm), (Aa, Ba, wa))

    parts = []
    for lo, hi in split:
        e0, e1 = lo * NW * C, hi * NW * C
        ep = hi - lo
        e_h = lax.slice_in_dim(edge, e0, e1, axis=0)
        src3d = e_h[:, 0].reshape(NW, ep, C)
        dst3d = e_h[:, 1].reshape(NW, ep, C)
        dst3s = e_h[:, 1].reshape(NW, ep * 2, C // 2)
        ef_h = lax.slice_in_dim(edge_feat, e0, e1, axis=0)
        Gs, Gd = _gather_diff(P, src3d, dst3d)
        m = _edge_mlp(Gs, Gd, ef_h, Wefm, msg_b1[None, :], Wefa,
                      att_b1[None, :], msg_W2, msg_b2[None, :], att_W2,
                      att_b2[None, :])
        parts.append(_scatter_add(m, dst3s, N))
    return _gru(parts[0], parts[1], node_feat, gru_Wih, gru_Whh,
                gru_bih[None, :], gru_bhh[None, :])


# speedup vs baseline: 1.0297x; 1.0297x over previous
import functools

import jax
import jax.numpy as jnp
from jax import lax
from jax.experimental import pallas as pl
from jax.experimental.pallas import tpu as pltpu
from jax.experimental.pallas import tpu_sc as plsc

NC = 2
NS = 16
NW = NC * NS


def _node_table(state, eattr, nattr, A, B):
    N, D = state.shape
    D2 = D
    BN = 1000

    def pack16(x):
        f = x.astype(jnp.bfloat16).astype(jnp.float32)
        return jax.lax.bitcast_convert_type(f, jnp.int32)

    def body(s_ref, e_ref, na_ref, Am_ref, Bm_ref, wm_ref, Aa_ref, Ba_ref,
             wa_ref, P_ref):
        u = s_ref[...] @ Am_ref[...] + e_ref[...] @ Bm_ref[...] \
            + na_ref[...] * wm_ref[...]
        v = s_ref[...] @ Aa_ref[...] + e_ref[...] @ Ba_ref[...] \
            + na_ref[...] * wa_ref[...]
        uw = jnp.bitwise_and(jnp.right_shift(pack16(u), 16), jnp.int32(65535))
        vw = jnp.bitwise_and(pack16(v), jnp.int32(-65536))
        P_ref[...] = jnp.bitwise_or(uw, vw)

    wspec = [
        pl.BlockSpec((D, D2), lambda i: (0, 0)),
        pl.BlockSpec((D, D2), lambda i: (0, 0)),
        pl.BlockSpec((1, D2), lambda i: (0, 0)),
    ]
    return pl.pallas_call(
        body,
        grid=(N // BN,),
        in_specs=[
            pl.BlockSpec((BN, D), lambda i: (i, 0)),
            pl.BlockSpec((BN, D), lambda i: (i, 0)),
            pl.BlockSpec((BN, 1), lambda i: (i, 0)),
        ] + wspec + wspec,
        out_specs=pl.BlockSpec((BN, D2), lambda i: (i, 0)),
        out_shape=jax.ShapeDtypeStruct((N, D2), jnp.int32),
    )(state, eattr, nattr, *A, *B)


def _gather_diff(P, src3d, dst3d):
    N, D2 = P.shape
    _, NCH, C = src3d.shape
    E = NW * NCH * C
    NB = 4
    OFF = 2
    NCYC = (NCH + OFF + NB - 1) // NB

    mesh = plsc.VectorSubcoreMesh(core_axis_name="c", subcore_axis_name="s")

    @functools.partial(
        pl.kernel,
        out_type=(jax.ShapeDtypeStruct((E, D2), jnp.int32),
                  jax.ShapeDtypeStruct((E, D2), jnp.int32)),
        mesh=mesh,
        scratch_types=[
            pltpu.VMEM((NCH, C), jnp.int32),
            pltpu.VMEM((NCH, C), jnp.int32),
            pltpu.VMEM((NB, C, D2), jnp.int32),
            pltpu.VMEM((NB, C, D2), jnp.int32),
        ] + [pltpu.SemaphoreType.DMA] * (2 * NB),
    )
    def gather_k(P_hbm, src_hbm, dst_hbm, outs_hbm, outd_hbm, sbuf, dbuf,
                 bufA, bufB, *sems):
        semg = sems[:NB]
        semw = sems[NB:]
        wid = lax.axis_index("s") * NC + lax.axis_index("c")
        crow0 = wid * NCH

        pltpu.sync_copy(src_hbm.at[wid], sbuf)
        pltpu.sync_copy(dst_hbm.at[wid], dbuf)

        def drain_wb(b):
            pltpu.make_async_copy(bufA.at[b], outs_hbm.at[pl.ds(0, C)],
                                  semw[b]).wait()
            pltpu.make_async_copy(bufB.at[b], outd_hbm.at[pl.ds(0, C)],
                                  semw[b]).wait()

        def cycle(g, _):
            for b in range(NB):
                j = g * NB + b
                bw = (b - OFF) % NB
                jw = g * NB + b - OFF

                @pl.when(j < NCH)
                def _():
                    @pl.when(j >= NB)
                    def _():
                        drain_wb(b)
                    pltpu.async_copy(P_hbm.at[sbuf.at[j]], bufA.at[b], semg[b])
                    pltpu.async_copy(P_hbm.at[dbuf.at[j]], bufB.at[b], semg[b])

                @pl.when(jnp.logical_and(jw >= 0, jw < NCH))
                def _():
                    pltpu.make_async_copy(P_hbm.at[sbuf.at[bw]], bufA.at[bw],
                                          semg[bw]).wait()
                    pltpu.make_async_copy(P_hbm.at[dbuf.at[bw]], bufB.at[bw],
                                          semg[bw]).wait()
                    rows = pl.ds((crow0 + jw) * C, C)
                    pltpu.async_copy(bufA.at[bw], outs_hbm.at[rows], semw[bw])
                    pltpu.async_copy(bufB.at[bw], outd_hbm.at[rows], semw[bw])
            return 0

        lax.fori_loop(0, NCYC, cycle, 0)
        for b in range(NB):
            drain_wb(b)

    return gather_k(P, src3d, dst3d)


def _edge_mlp(Gs, Gd, eft, off, Wefm, b1m, Wefa, b1a, W2m, b2m, W2a, b2a):
    E, D = Gs.shape
    DE = eft.shape[0]
    BE = 3200
    bf = jnp.bfloat16
    cdim = (((0,), (0,)), ((), ()))

    def unpack(g):
        lo = jax.lax.bitcast_convert_type(jnp.left_shift(g, 16), jnp.float32)
        hi = jax.lax.bitcast_convert_type(
            jnp.bitwise_and(g, jnp.int32(-65536)), jnp.float32)
        return lo, hi

    def body(Gs_ref, Gd_ref, ef_ref, Wefm_ref, b1m_ref, Wefa_ref, b1a_ref,
             W2m_ref, b2m_ref, W2a_ref, b2a_ref, o_ref):
        sm, sa = unpack(Gs_ref[...])
        dm, da = unpack(Gd_ref[...])
        gm = sm - dm
        ga = sa - da
        efv = ef_ref[...]
        em = jax.lax.dot_general(efv, Wefm_ref[...], cdim,
                                 preferred_element_type=jnp.float32)
        ea = jax.lax.dot_general(efv, Wefa_ref[...], cdim,
                                 preferred_element_type=jnp.float32)
        hm = jnp.maximum(gm + em + b1m_ref[...], 0.0).astype(bf)
        ha = jnp.maximum(ga + ea + b1a_ref[...], 0.0).astype(bf)
        msg = jax.lax.dot(hm, W2m_ref[...].astype(bf),
                          preferred_element_type=jnp.float32) + b2m_ref[...]
        att = jax.lax.dot(ha, W2a_ref[...].astype(bf),
                          preferred_element_type=jnp.float32) + b2a_ref[...]
        o_ref[...] = msg * jax.nn.sigmoid(att)

    ob = off // BE
    wspec = [
        pl.BlockSpec((DE, D), lambda i: (0, 0)),
        pl.BlockSpec((1, D), lambda i: (0, 0)),
    ]
    return pl.pallas_call(
        body,
        grid=(E // BE,),
        in_specs=[
            pl.BlockSpec((BE, D), lambda i: (i, 0)),
            pl.BlockSpec((BE, D), lambda i: (i, 0)),
            pl.BlockSpec((DE, BE), lambda i: (0, i + ob)),
        ] + wspec + wspec + [
            pl.BlockSpec((D, D), lambda i: (0, 0)),
            pl.BlockSpec((1, D), lambda i: (0, 0)),
            pl.BlockSpec((D, D), lambda i: (0, 0)),
            pl.BlockSpec((1, D), lambda i: (0, 0)),
        ],
        out_specs=pl.BlockSpec((BE, D), lambda i: (i, 0)),
        out_shape=jax.ShapeDtypeStruct((E, D), jnp.float32),
    )(Gs, Gd, eft, Wefm, b1m, Wefa, b1a, W2m, b2m, W2a, b2a)


def _scatter_add(m, dst3d, N):
    E, D = m.shape
    _, NCH, C2 = dst3d.shape
    M = 3
    NCYC = (NCH + M - 1) // M
    CZ = 16
    NZCH = N // CZ
    ZPT = (NZCH + NS - 1) // NS

    mesh = plsc.VectorSubcoreMesh(core_axis_name="c", subcore_axis_name="s")

    @functools.partial(
        pl.kernel,
        out_type=jax.ShapeDtypeStruct((NC, N, D), jnp.float32),
        mesh=mesh,
        scratch_types=[
            pltpu.VMEM_SHARED((N, D), jnp.float32),
            pltpu.VMEM((NCH, C2), jnp.int32),
            pltpu.VMEM((M, C2, D), jnp.float32),
            pltpu.VMEM((CZ, D), jnp.float32),
        ] + [pltpu.SemaphoreType.DMA] * (2 * M),
    )
    def scatter_k(m_hbm, dst_hbm, out_hbm, acc, dbuf, mbuf, zbuf, *sems):
        semL = sems[:M]
        semS = sems[M:]
        c = lax.axis_index("c")
        s = lax.axis_index("s")
        wid = s * NC + c
        crow0 = wid * NCH

        def zrow(r, _):
            for k in range(D // 16):
                zbuf[r, pl.ds(k * 16, 16)] = jnp.zeros((16,), jnp.float32)
            return 0

        lax.fori_loop(0, CZ, zrow, 0)

        def zchunk(kk, _):
            jz = kk * NS + s

            @pl.when(jz < NZCH)
            def _():
                pltpu.sync_copy(zbuf, acc.at[pl.ds(jz * CZ, CZ)])
            return 0

        lax.fori_loop(0, ZPT, zchunk, 0)
        pltpu.sync_copy(dst_hbm.at[wid], dbuf)
        plsc.subcore_barrier()

        def cycle(g, _):
            for b in range(M):
                j = g * M + b
                bs = (b - M // 2) % M
                js = j - M // 2

                @pl.when(j < NCH)
                def _():
                    @pl.when(j >= M)
                    def _():
                        pltpu.make_async_copy(
                            mbuf.at[b], acc.at[dbuf.at[0]], semS[b]).wait()
                    pltpu.async_copy(m_hbm.at[pl.ds((crow0 + j) * C2, C2)],
                                     mbuf.at[b], semL[b])

                @pl.when(jnp.logical_and(js >= 0, js < NCH))
                def _():
                    pltpu.make_async_copy(
                        m_hbm.at[pl.ds(0, C2)], mbuf.at[bs], semL[bs]).wait()
                    pltpu.async_copy(mbuf.at[bs], acc.at[dbuf.at[js]], semS[bs],
                                     add=True)
            return 0

        lax.fori_loop(0, NCYC + 1, cycle, 0)
        for b in range(M):
            pltpu.make_async_copy(mbuf.at[b], acc.at[dbuf.at[0]], semS[b]).wait()
        plsc.subcore_barrier()

        def dchunk(kk, _):
            jz = kk * NS + s

            @pl.when(jz < NZCH)
            def _():
                rows = pl.ds(jz * CZ, CZ)
                pltpu.sync_copy(acc.at[rows], zbuf)
                pltpu.sync_copy(zbuf, out_hbm.at[c, rows])
            return 0

        lax.fori_loop(0, ZPT, dchunk, 0)

    return scatter_k(m, dst3d)


def _gru(parts, state, Wih, Whh, bih, bhh):
    N, D = state.shape
    D3 = Wih.shape[1]
    K = len(parts)
    BN = 1000

    def body(*refs):
        p_refs = refs[:K]
        s_ref, Wih_ref, Whh_ref, bih_ref, bhh_ref, o_ref = refs[K:]
        x = p_refs[0][0] + p_refs[0][1]
        for pr in p_refs[1:]:
            x = x + pr[0] + pr[1]
        h = s_ref[...]
        gi = x @ Wih_ref[...] + bih_ref[...]
        gh = h @ Whh_ref[...] + bhh_ref[...]
        r = jax.nn.sigmoid(gi[:, :D] + gh[:, :D])
        z = jax.nn.sigmoid(gi[:, D:2 * D] + gh[:, D:2 * D])
        n = jnp.tanh(gi[:, 2 * D:] + r * gh[:, 2 * D:])
        o_ref[...] = (1.0 - z) * n + z * h

    return pl.pallas_call(
        body,
        grid=(N // BN,),
        in_specs=[pl.BlockSpec((2, BN, D), lambda i: (0, i, 0))] * K + [
            pl.BlockSpec((BN, D), lambda i: (i, 0)),
            pl.BlockSpec((D, D3), lambda i: (0, 0)),
            pl.BlockSpec((D, D3), lambda i: (0, 0)),
            pl.BlockSpec((1, D3), lambda i: (0, 0)),
            pl.BlockSpec((1, D3), lambda i: (0, 0)),
        ],
        out_specs=pl.BlockSpec((BN, D), lambda i: (i, 0)),
        out_shape=jax.ShapeDtypeStruct((N, D), jnp.float32),
    )(*parts, state, Wih, Whh, bih, bhh)


def kernel(node_feat, edge, edge_feat, node_attributes, edge_attributes,
           msg_W1, msg_b1, msg_W2, msg_b2, att_W1, att_b1, att_W2, att_b2,
           gru_Wih, gru_Whh, gru_bih, gru_bhh):
    N, D = node_feat.shape
    E = edge.shape[0]
    DE = edge_feat.shape[1]
    C = 40
    K = 5
    EP = E // K
    eattr = edge_attributes[0]
    eft = edge_feat.T
    nattr = node_attributes[0][:, None]

    Am, Wefm, Bm, wm = (msg_W1[:D], msg_W1[D:D + DE],
                        msg_W1[D + DE:D + DE + D], msg_W1[D + DE + D:])
    Aa, Wefa, Ba, wa = (att_W1[:D], att_W1[D:D + DE],
                        att_W1[D + DE:D + DE + D], att_W1[D + DE + D:])

    P = _node_table(node_feat, eattr, nattr, (Am, Bm, wm), (Aa, Ba, wa))

    parts = []
    for h in range(K):
        e_h = lax.slice_in_dim(edge, h * EP, (h + 1) * EP, axis=0)
        src3d = e_h[:, 0].reshape(NW, EP // (NW * C), C)
        dst3d = e_h[:, 1].reshape(NW, EP // (NW * C), C)
        Gs, Gd = _gather_diff(P, src3d, dst3d)
        m = _edge_mlp(Gs, Gd, eft, h * EP, Wefm, msg_b1[None, :], Wefa,
                      att_b1[None, :], msg_W2, msg_b2[None, :], att_W2,
                      att_b2[None, :])
        parts.append(_scatter_add(m, dst3d, N))
    return _gru(parts, node_feat, gru_Wih, gru_Whh,
                gru_bih[None, :], gru_bhh[None, :])


# speedup vs baseline: 1.1466x; 1.1135x over previous
import functools

import jax
import jax.numpy as jnp
from jax import lax
from jax.experimental import pallas as pl
from jax.experimental.pallas import tpu as pltpu
from jax.experimental.pallas import tpu_sc as plsc

NC = 2
NS = 16
NW = NC * NS


def _node_table(state, eattr, nattr, A, B):
    N, D = state.shape
    D2 = D
    BN = 1000

    def pack16(x):
        f = x.astype(jnp.bfloat16).astype(jnp.float32)
        return jax.lax.bitcast_convert_type(f, jnp.int32)

    def body(s_ref, e_ref, na_ref, Am_ref, Bm_ref, wm_ref, Aa_ref, Ba_ref,
             wa_ref, P_ref):
        u = s_ref[...] @ Am_ref[...] + e_ref[...] @ Bm_ref[...] \
            + na_ref[...] * wm_ref[...]
        v = s_ref[...] @ Aa_ref[...] + e_ref[...] @ Ba_ref[...] \
            + na_ref[...] * wa_ref[...]
        uw = jnp.bitwise_and(jnp.right_shift(pack16(u), 16), jnp.int32(65535))
        vw = jnp.bitwise_and(pack16(v), jnp.int32(-65536))
        P_ref[...] = jnp.bitwise_or(uw, vw)

    wspec = [
        pl.BlockSpec((D, D2), lambda i: (0, 0)),
        pl.BlockSpec((D, D2), lambda i: (0, 0)),
        pl.BlockSpec((1, D2), lambda i: (0, 0)),
    ]
    return pl.pallas_call(
        body,
        grid=(N // BN,),
        in_specs=[
            pl.BlockSpec((BN, D), lambda i: (i, 0)),
            pl.BlockSpec((BN, D), lambda i: (i, 0)),
            pl.BlockSpec((BN, 1), lambda i: (i, 0)),
        ] + wspec + wspec,
        out_specs=pl.BlockSpec((BN, D2), lambda i: (i, 0)),
        out_shape=jax.ShapeDtypeStruct((N, D2), jnp.int32),
    )(state, eattr, nattr, *A, *B)


def _gather_diff(P, src3d, dst3d):
    N, D2 = P.shape
    _, NCH, C = src3d.shape
    E = NW * NCH * C
    NB = 4
    OFF = 2
    NCYC = (NCH + OFF + NB - 1) // NB

    mesh = plsc.VectorSubcoreMesh(core_axis_name="c", subcore_axis_name="s")

    @functools.partial(
        pl.kernel,
        out_type=(jax.ShapeDtypeStruct((E, D2), jnp.int32),
                  jax.ShapeDtypeStruct((E, D2), jnp.int32)),
        mesh=mesh,
        scratch_types=[
            pltpu.VMEM((NCH, C), jnp.int32),
            pltpu.VMEM((NCH, C), jnp.int32),
            pltpu.VMEM((NB, C, D2), jnp.int32),
            pltpu.VMEM((NB, C, D2), jnp.int32),
        ] + [pltpu.SemaphoreType.DMA] * (2 * NB),
    )
    def gather_k(P_hbm, src_hbm, dst_hbm, outs_hbm, outd_hbm, sbuf, dbuf,
                 bufA, bufB, *sems):
        semg = sems[:NB]
        semw = sems[NB:]
        wid = lax.axis_index("s") * NC + lax.axis_index("c")
        crow0 = wid * NCH

        pltpu.sync_copy(src_hbm.at[wid], sbuf)
        pltpu.sync_copy(dst_hbm.at[wid], dbuf)

        def drain_wb(b):
            pltpu.make_async_copy(bufA.at[b], outs_hbm.at[pl.ds(0, C)],
                                  semw[b]).wait()
            pltpu.make_async_copy(bufB.at[b], outd_hbm.at[pl.ds(0, C)],
                                  semw[b]).wait()

        def cycle(g, _):
            for b in range(NB):
                j = g * NB + b
                bw = (b - OFF) % NB
                jw = g * NB + b - OFF

                @pl.when(j < NCH)
                def _():
                    @pl.when(j >= NB)
                    def _():
                        drain_wb(b)
                    pltpu.async_copy(P_hbm.at[sbuf.at[j]], bufA.at[b], semg[b])
                    pltpu.async_copy(P_hbm.at[dbuf.at[j]], bufB.at[b], semg[b])

                @pl.when(jnp.logical_and(jw >= 0, jw < NCH))
                def _():
                    pltpu.make_async_copy(P_hbm.at[sbuf.at[bw]], bufA.at[bw],
                                          semg[bw]).wait()
                    pltpu.make_async_copy(P_hbm.at[dbuf.at[bw]], bufB.at[bw],
                                          semg[bw]).wait()
                    rows = pl.ds((crow0 + jw) * C, C)
                    pltpu.async_copy(bufA.at[bw], outs_hbm.at[rows], semw[bw])
                    pltpu.async_copy(bufB.at[bw], outd_hbm.at[rows], semw[bw])
            return 0

        lax.fori_loop(0, NCYC, cycle, 0)
        for b in range(NB):
            drain_wb(b)

    return gather_k(P, src3d, dst3d)


def _edge_mlp(Gs, Gd, eft, off, Wefm, b1m, Wefa, b1a, W2m, b2m, W2a, b2a):
    E, D = Gs.shape
    DE = eft.shape[0]
    BE = 3200
    bf = jnp.bfloat16
    cdim = (((0,), (0,)), ((), ()))

    def unpack(g):
        lo = jax.lax.bitcast_convert_type(jnp.left_shift(g, 16), jnp.float32)
        hi = jax.lax.bitcast_convert_type(
            jnp.bitwise_and(g, jnp.int32(-65536)), jnp.float32)
        return lo, hi

    def body(Gs_ref, Gd_ref, ef_ref, Wefm_ref, b1m_ref, Wefa_ref, b1a_ref,
             W2m_ref, b2m_ref, W2a_ref, b2a_ref, o_ref):
        sm, sa = unpack(Gs_ref[...])
        dm, da = unpack(Gd_ref[...])
        gm = sm - dm
        ga = sa - da
        efv = ef_ref[...]
        em = jax.lax.dot_general(efv, Wefm_ref[...], cdim,
                                 preferred_element_type=jnp.float32)
        ea = jax.lax.dot_general(efv, Wefa_ref[...], cdim,
                                 preferred_element_type=jnp.float32)
        hm = jnp.maximum(gm + em + b1m_ref[...], 0.0).astype(bf)
        ha = jnp.maximum(ga + ea + b1a_ref[...], 0.0).astype(bf)
        msg = jax.lax.dot(hm, W2m_ref[...].astype(bf),
                          preferred_element_type=jnp.float32) + b2m_ref[...]
        att = jax.lax.dot(ha, W2a_ref[...].astype(bf),
                          preferred_element_type=jnp.float32) + b2a_ref[...]
        o_ref[...] = msg * jax.nn.sigmoid(att)

    ob = off // BE
    wspec = [
        pl.BlockSpec((DE, D), lambda i: (0, 0)),
        pl.BlockSpec((1, D), lambda i: (0, 0)),
    ]
    return pl.pallas_call(
        body,
        grid=(E // BE,),
        in_specs=[
            pl.BlockSpec((BE, D), lambda i: (i, 0)),
            pl.BlockSpec((BE, D), lambda i: (i, 0)),
            pl.BlockSpec((DE, BE), lambda i: (0, i + ob)),
        ] + wspec + wspec + [
            pl.BlockSpec((D, D), lambda i: (0, 0)),
            pl.BlockSpec((1, D), lambda i: (0, 0)),
            pl.BlockSpec((D, D), lambda i: (0, 0)),
            pl.BlockSpec((1, D), lambda i: (0, 0)),
        ],
        out_specs=pl.BlockSpec((BE, D), lambda i: (i, 0)),
        out_shape=jax.ShapeDtypeStruct((E, D), jnp.float32),
    )(Gs, Gd, eft, Wefm, b1m, Wefa, b1a, W2m, b2m, W2a, b2a)


def _scatter_add(ms, dsts, N):
    K = len(ms)
    E, D = ms[0].shape
    _, NCH, C2 = dsts[0].shape
    M = 3
    NCYC = (NCH + M - 1) // M
    CZ = 16
    NZCH = N // CZ
    ZPT = (NZCH + NS - 1) // NS

    mesh = plsc.VectorSubcoreMesh(core_axis_name="c", subcore_axis_name="s")

    @functools.partial(
        pl.kernel,
        out_type=jax.ShapeDtypeStruct((NC, N, D), jnp.float32),
        mesh=mesh,
        scratch_types=[
            pltpu.VMEM_SHARED((N, D), jnp.float32),
            pltpu.VMEM((K * NCH, C2), jnp.int32),
            pltpu.VMEM((M, C2, D), jnp.float32),
            pltpu.VMEM((CZ, D), jnp.float32),
        ] + [pltpu.SemaphoreType.DMA] * (2 * M),
    )
    def scatter_k(*refs):
        m_hbms = refs[:K]
        d_hbms = refs[K:2 * K]
        out_hbm = refs[2 * K]
        acc, dbuf, mbuf, zbuf = refs[2 * K + 1:2 * K + 5]
        sems = refs[2 * K + 5:]
        semL = sems[:M]
        semS = sems[M:]
        c = lax.axis_index("c")
        s = lax.axis_index("s")
        wid = s * NC + c
        crow0 = wid * NCH

        def zrow(r, _):
            for k in range(D // 16):
                zbuf[r, pl.ds(k * 16, 16)] = jnp.zeros((16,), jnp.float32)
            return 0

        lax.fori_loop(0, CZ, zrow, 0)

        def zchunk(kk, _):
            jz = kk * NS + s

            @pl.when(jz < NZCH)
            def _():
                pltpu.sync_copy(zbuf, acc.at[pl.ds(jz * CZ, CZ)])
            return 0

        lax.fori_loop(0, ZPT, zchunk, 0)
        for h in range(K):
            pltpu.sync_copy(d_hbms[h].at[wid], dbuf.at[pl.ds(h * NCH, NCH)])
        plsc.subcore_barrier()

        for h in range(K):
            m_hbm = m_hbms[h]

            def cycle(g, _):
                for b in range(M):
                    j = g * M + b
                    bs = (b - M // 2) % M
                    js = j - M // 2

                    @pl.when(j < NCH)
                    def _():
                        @pl.when(j >= M)
                        def _():
                            pltpu.make_async_copy(
                                mbuf.at[b], acc.at[dbuf.at[0]], semS[b]).wait()
                        pltpu.async_copy(m_hbm.at[pl.ds((crow0 + j) * C2, C2)],
                                         mbuf.at[b], semL[b])

                    @pl.when(jnp.logical_and(js >= 0, js < NCH))
                    def _():
                        pltpu.make_async_copy(
                            m_hbm.at[pl.ds(0, C2)], mbuf.at[bs], semL[bs]).wait()
                        pltpu.async_copy(mbuf.at[bs],
                                         acc.at[dbuf.at[h * NCH + js]], semS[bs],
                                         add=True)
                return 0

            lax.fori_loop(0, NCYC + 1, cycle, 0)
            for b in range(M):
                pltpu.make_async_copy(mbuf.at[b], acc.at[dbuf.at[0]],
                                      semS[b]).wait()
        plsc.subcore_barrier()

        def dchunk(kk, _):
            jz = kk * NS + s

            @pl.when(jz < NZCH)
            def _():
                rows = pl.ds(jz * CZ, CZ)
                pltpu.sync_copy(acc.at[rows], zbuf)
                pltpu.sync_copy(zbuf, out_hbm.at[c, rows])
            return 0

        lax.fori_loop(0, ZPT, dchunk, 0)

    return scatter_k(*ms, *dsts)


def _gru(parts, state, Wih, Whh, bih, bhh):
    N, D = state.shape
    D3 = Wih.shape[1]
    K = len(parts)
    BN = 1000

    def body(*refs):
        p_refs = refs[:K]
        s_ref, Wih_ref, Whh_ref, bih_ref, bhh_ref, o_ref = refs[K:]
        x = p_refs[0][0] + p_refs[0][1]
        for pr in p_refs[1:]:
            x = x + pr[0] + pr[1]
        h = s_ref[...]
        gi = x @ Wih_ref[...] + bih_ref[...]
        gh = h @ Whh_ref[...] + bhh_ref[...]
        r = jax.nn.sigmoid(gi[:, :D] + gh[:, :D])
        z = jax.nn.sigmoid(gi[:, D:2 * D] + gh[:, D:2 * D])
        n = jnp.tanh(gi[:, 2 * D:] + r * gh[:, 2 * D:])
        o_ref[...] = (1.0 - z) * n + z * h

    return pl.pallas_call(
        body,
        grid=(N // BN,),
        in_specs=[pl.BlockSpec((2, BN, D), lambda i: (0, i, 0))] * K + [
            pl.BlockSpec((BN, D), lambda i: (i, 0)),
            pl.BlockSpec((D, D3), lambda i: (0, 0)),
            pl.BlockSpec((D, D3), lambda i: (0, 0)),
            pl.BlockSpec((1, D3), lambda i: (0, 0)),
            pl.BlockSpec((1, D3), lambda i: (0, 0)),
        ],
        out_specs=pl.BlockSpec((BN, D), lambda i: (i, 0)),
        out_shape=jax.ShapeDtypeStruct((N, D), jnp.float32),
    )(*parts, state, Wih, Whh, bih, bhh)


def kernel(node_feat, edge, edge_feat, node_attributes, edge_attributes,
           msg_W1, msg_b1, msg_W2, msg_b2, att_W1, att_b1, att_W2, att_b2,
           gru_Wih, gru_Whh, gru_bih, gru_bhh):
    N, D = node_feat.shape
    E = edge.shape[0]
    DE = edge_feat.shape[1]
    C = 40
    K = 5
    EP = E // K
    eattr = edge_attributes[0]
    eft = edge_feat.T
    nattr = node_attributes[0][:, None]

    Am, Wefm, Bm, wm = (msg_W1[:D], msg_W1[D:D + DE],
                        msg_W1[D + DE:D + DE + D], msg_W1[D + DE + D:])
    Aa, Wefa, Ba, wa = (att_W1[:D], att_W1[D:D + DE],
                        att_W1[D + DE:D + DE + D], att_W1[D + DE + D:])

    P = _node_table(node_feat, eattr, nattr, (Am, Bm, wm), (Aa, Ba, wa))

    ms, dsts = [], []
    for h in range(K):
        e_h = lax.slice_in_dim(edge, h * EP, (h + 1) * EP, axis=0)
        src3d = e_h[:, 0].reshape(NW, EP // (NW * C), C)
        dst3d = e_h[:, 1].reshape(NW, EP // (NW * C), C)
        Gs, Gd = _gather_diff(P, src3d, dst3d)
        m = _edge_mlp(Gs, Gd, eft, h * EP, Wefm, msg_b1[None, :], Wefa,
                      att_b1[None, :], msg_W2, msg_b2[None, :], att_W2,
                      att_b2[None, :])
        ms.append(m)
        dsts.append(dst3d)
    parts = _scatter_add(ms, dsts, N)
    return _gru([parts], node_feat, gru_Wih, gru_Whh,
                gru_bih[None, :], gru_bhh[None, :])


# speedup vs baseline: 1.1475x; 1.0008x over previous
import functools

import jax
import jax.numpy as jnp
from jax import lax
from jax.experimental import pallas as pl
from jax.experimental.pallas import tpu as pltpu
from jax.experimental.pallas import tpu_sc as plsc

NC = 2
NS = 16
NW = NC * NS


def _node_table(state, eattr, nattr, A, B):
    N, D = state.shape
    D2 = D
    BN = 1000

    def pack16(x):
        f = x.astype(jnp.bfloat16).astype(jnp.float32)
        return jax.lax.bitcast_convert_type(f, jnp.int32)

    def body(s_ref, e_ref, na_ref, Am_ref, Bm_ref, wm_ref, Aa_ref, Ba_ref,
             wa_ref, P_ref):
        u = s_ref[...] @ Am_ref[...] + e_ref[...] @ Bm_ref[...] \
            + na_ref[...] * wm_ref[...]
        v = s_ref[...] @ Aa_ref[...] + e_ref[...] @ Ba_ref[...] \
            + na_ref[...] * wa_ref[...]
        uw = jnp.bitwise_and(jnp.right_shift(pack16(u), 16), jnp.int32(65535))
        vw = jnp.bitwise_and(pack16(v), jnp.int32(-65536))
        P_ref[...] = jnp.bitwise_or(uw, vw)

    wspec = [
        pl.BlockSpec((D, D2), lambda i: (0, 0)),
        pl.BlockSpec((D, D2), lambda i: (0, 0)),
        pl.BlockSpec((1, D2), lambda i: (0, 0)),
    ]
    return pl.pallas_call(
        body,
        grid=(N // BN,),
        in_specs=[
            pl.BlockSpec((BN, D), lambda i: (i, 0)),
            pl.BlockSpec((BN, D), lambda i: (i, 0)),
            pl.BlockSpec((BN, 1), lambda i: (i, 0)),
        ] + wspec + wspec,
        out_specs=pl.BlockSpec((BN, D2), lambda i: (i, 0)),
        out_shape=jax.ShapeDtypeStruct((N, D2), jnp.int32),
    )(state, eattr, nattr, *A, *B)


def _gather_diff(P, src3d, dst3d):
    N, D2 = P.shape
    _, NCH, C = src3d.shape
    E = NW * NCH * C
    NB = 4
    OFF = 2
    NCYC = (NCH + OFF + NB - 1) // NB

    mesh = plsc.VectorSubcoreMesh(core_axis_name="c", subcore_axis_name="s")

    @functools.partial(
        pl.kernel,
        out_type=(jax.ShapeDtypeStruct((E, D2), jnp.int32),
                  jax.ShapeDtypeStruct((E, D2), jnp.int32)),
        mesh=mesh,
        scratch_types=[
            pltpu.VMEM((NCH, C), jnp.int32),
            pltpu.VMEM((NCH, C), jnp.int32),
            pltpu.VMEM((NB, C, D2), jnp.int32),
            pltpu.VMEM((NB, C, D2), jnp.int32),
        ] + [pltpu.SemaphoreType.DMA] * (2 * NB),
    )
    def gather_k(P_hbm, src_hbm, dst_hbm, outs_hbm, outd_hbm, sbuf, dbuf,
                 bufA, bufB, *sems):
        semg = sems[:NB]
        semw = sems[NB:]
        wid = lax.axis_index("s") * NC + lax.axis_index("c")
        crow0 = wid * NCH

        pltpu.sync_copy(src_hbm.at[wid], sbuf)
        pltpu.sync_copy(dst_hbm.at[wid], dbuf)

        def drain_wb(b):
            pltpu.make_async_copy(bufA.at[b], outs_hbm.at[pl.ds(0, C)],
                                  semw[b]).wait()
            pltpu.make_async_copy(bufB.at[b], outd_hbm.at[pl.ds(0, C)],
                                  semw[b]).wait()

        def cycle(g, _):
            for b in range(NB):
                j = g * NB + b
                bw = (b - OFF) % NB
                jw = g * NB + b - OFF

                @pl.when(j < NCH)
                def _():
                    @pl.when(j >= NB)
                    def _():
                        drain_wb(b)
                    pltpu.async_copy(P_hbm.at[sbuf.at[j]], bufA.at[b], semg[b])
                    pltpu.async_copy(P_hbm.at[dbuf.at[j]], bufB.at[b], semg[b])

                @pl.when(jnp.logical_and(jw >= 0, jw < NCH))
                def _():
                    pltpu.make_async_copy(P_hbm.at[sbuf.at[bw]], bufA.at[bw],
                                          semg[bw]).wait()
                    pltpu.make_async_copy(P_hbm.at[dbuf.at[bw]], bufB.at[bw],
                                          semg[bw]).wait()
                    rows = pl.ds((crow0 + jw) * C, C)
                    pltpu.async_copy(bufA.at[bw], outs_hbm.at[rows], semw[bw])
                    pltpu.async_copy(bufB.at[bw], outd_hbm.at[rows], semw[bw])
            return 0

        lax.fori_loop(0, NCYC, cycle, 0)
        for b in range(NB):
            drain_wb(b)

    return gather_k(P, src3d, dst3d)


def _edge_mlp(Gs, Gd, eft, off, Wefm, b1m, Wefa, b1a, W2m, b2m, W2a, b2a):
    E, D = Gs.shape
    DE = eft.shape[0]
    BE = 3200
    bf = jnp.bfloat16
    cdim = (((0,), (0,)), ((), ()))

    def unpack(g):
        lo = jax.lax.bitcast_convert_type(jnp.left_shift(g, 16), jnp.float32)
        hi = jax.lax.bitcast_convert_type(
            jnp.bitwise_and(g, jnp.int32(-65536)), jnp.float32)
        return lo, hi

    def body(Gs_ref, Gd_ref, ef_ref, Wefm_ref, b1m_ref, Wefa_ref, b1a_ref,
             W2m_ref, b2m_ref, W2a_ref, b2a_ref, o_ref):
        sm, sa = unpack(Gs_ref[...])
        dm, da = unpack(Gd_ref[...])
        gm = sm - dm
        ga = sa - da
        efv = ef_ref[...]
        em = jax.lax.dot_general(efv, Wefm_ref[...], cdim,
                                 preferred_element_type=jnp.float32)
        ea = jax.lax.dot_general(efv, Wefa_ref[...], cdim,
                                 preferred_element_type=jnp.float32)
        hm = jnp.maximum(gm + em + b1m_ref[...], 0.0).astype(bf)
        ha = jnp.maximum(ga + ea + b1a_ref[...], 0.0).astype(bf)
        msg = jax.lax.dot(hm, W2m_ref[...].astype(bf),
                          preferred_element_type=jnp.float32) + b2m_ref[...]
        att = jax.lax.dot(ha, W2a_ref[...].astype(bf),
                          preferred_element_type=jnp.float32) + b2a_ref[...]
        o_ref[...] = msg * jax.nn.sigmoid(att)

    ob = off // BE
    wspec = [
        pl.BlockSpec((DE, D), lambda i: (0, 0)),
        pl.BlockSpec((1, D), lambda i: (0, 0)),
    ]
    return pl.pallas_call(
        body,
        grid=(E // BE,),
        in_specs=[
            pl.BlockSpec((BE, D), lambda i: (i, 0)),
            pl.BlockSpec((BE, D), lambda i: (i, 0)),
            pl.BlockSpec((DE, BE), lambda i: (0, i + ob)),
        ] + wspec + wspec + [
            pl.BlockSpec((D, D), lambda i: (0, 0)),
            pl.BlockSpec((1, D), lambda i: (0, 0)),
            pl.BlockSpec((D, D), lambda i: (0, 0)),
            pl.BlockSpec((1, D), lambda i: (0, 0)),
        ],
        out_specs=pl.BlockSpec((BE, D), lambda i: (i, 0)),
        out_shape=jax.ShapeDtypeStruct((E, D), jnp.float32),
    )(Gs, Gd, eft, Wefm, b1m, Wefa, b1a, W2m, b2m, W2a, b2a)


def _scatter_add(ms, dsts, N):
    K = len(ms)
    E, D = ms[0].shape
    _, NCH, C2 = dsts[0].shape
    M = 3
    NCYC = (NCH + M - 1) // M
    CZ = 16
    NZCH = N // CZ
    ZPT = (NZCH + NS - 1) // NS

    mesh = plsc.VectorSubcoreMesh(core_axis_name="c", subcore_axis_name="s")

    @functools.partial(
        pl.kernel,
        out_type=jax.ShapeDtypeStruct((NC, N, D), jnp.float32),
        mesh=mesh,
        scratch_types=[
            pltpu.VMEM_SHARED((N, D), jnp.float32),
            pltpu.VMEM((K * NCH, C2), jnp.int32),
            pltpu.VMEM((M, C2, D), jnp.float32),
            pltpu.VMEM((CZ, D), jnp.float32),
        ] + [pltpu.SemaphoreType.DMA] * (2 * M),
    )
    def scatter_k(*refs):
        m_hbms = refs[:K]
        d_hbms = refs[K:2 * K]
        out_hbm = refs[2 * K]
        acc, dbuf, mbuf, zbuf = refs[2 * K + 1:2 * K + 5]
        sems = refs[2 * K + 5:]
        semL = sems[:M]
        semS = sems[M:]
        c = lax.axis_index("c")
        s = lax.axis_index("s")
        wid = s * NC + c
        crow0 = wid * NCH

        def zrow(r, _):
            for k in range(D // 16):
                zbuf[r, pl.ds(k * 16, 16)] = jnp.zeros((16,), jnp.float32)
            return 0

        lax.fori_loop(0, CZ, zrow, 0)

        def zchunk(kk, _):
            jz = kk * NS + s

            @pl.when(jz < NZCH)
            def _():
                pltpu.sync_copy(zbuf, acc.at[pl.ds(jz * CZ, CZ)])
            return 0

        lax.fori_loop(0, ZPT, zchunk, 0)
        for h in range(K):
            pltpu.sync_copy(d_hbms[h].at[wid], dbuf.at[pl.ds(h * NCH, NCH)])
        plsc.subcore_barrier()

        for h in range(K):
            m_hbm = m_hbms[h]

            def cycle(g, _):
                for b in range(M):
                    j = g * M + b
                    bs = (b - M // 2) % M
                    js = j - M // 2

                    @pl.when(j < NCH)
                    def _():
                        @pl.when(j >= M)
                        def _():
                            pltpu.make_async_copy(
                                mbuf.at[b], acc.at[dbuf.at[0]], semS[b]).wait()
                        pltpu.async_copy(m_hbm.at[pl.ds((crow0 + j) * C2, C2)],
                                         mbuf.at[b], semL[b])

                    @pl.when(jnp.logical_and(js >= 0, js < NCH))
                    def _():
                        pltpu.make_async_copy(
                            m_hbm.at[pl.ds(0, C2)], mbuf.at[bs], semL[bs]).wait()
                        pltpu.async_copy(mbuf.at[bs],
                                         acc.at[dbuf.at[h * NCH + js]], semS[bs],
                                         add=True)
                return 0

            lax.fori_loop(0, NCYC + 1, cycle, 0)
            for b in range(M):
                pltpu.make_async_copy(mbuf.at[b], acc.at[dbuf.at[0]],
                                      semS[b]).wait()
        plsc.subcore_barrier()

        def dchunk(kk, _):
            jz = kk * NS + s

            @pl.when(jz < NZCH)
            def _():
                rows = pl.ds(jz * CZ, CZ)
                pltpu.sync_copy(acc.at[rows], zbuf)
                pltpu.sync_copy(zbuf, out_hbm.at[c, rows])
            return 0

        lax.fori_loop(0, ZPT, dchunk, 0)

    return scatter_k(*ms, *dsts)


def _gru(parts, state, Wih, Whh, bih, bhh):
    N, D = state.shape
    D3 = Wih.shape[1]
    K = len(parts)
    BN = 1000

    def body(*refs):
        p_refs = refs[:K]
        s_ref, Wih_ref, Whh_ref, bih_ref, bhh_ref, o_ref = refs[K:]
        x = p_refs[0][0] + p_refs[0][1]
        for pr in p_refs[1:]:
            x = x + pr[0] + pr[1]
        h = s_ref[...]
        gi = x @ Wih_ref[...] + bih_ref[...]
        gh = h @ Whh_ref[...] + bhh_ref[...]
        r = jax.nn.sigmoid(gi[:, :D] + gh[:, :D])
        z = jax.nn.sigmoid(gi[:, D:2 * D] + gh[:, D:2 * D])
        n = jnp.tanh(gi[:, 2 * D:] + r * gh[:, 2 * D:])
        o_ref[...] = (1.0 - z) * n + z * h

    return pl.pallas_call(
        body,
        grid=(N // BN,),
        in_specs=[pl.BlockSpec((2, BN, D), lambda i: (0, i, 0))] * K + [
            pl.BlockSpec((BN, D), lambda i: (i, 0)),
            pl.BlockSpec((D, D3), lambda i: (0, 0)),
            pl.BlockSpec((D, D3), lambda i: (0, 0)),
            pl.BlockSpec((1, D3), lambda i: (0, 0)),
            pl.BlockSpec((1, D3), lambda i: (0, 0)),
        ],
        out_specs=pl.BlockSpec((BN, D), lambda i: (i, 0)),
        out_shape=jax.ShapeDtypeStruct((N, D), jnp.float32),
    )(*parts, state, Wih, Whh, bih, bhh)


def kernel(node_feat, edge, edge_feat, node_attributes, edge_attributes,
           msg_W1, msg_b1, msg_W2, msg_b2, att_W1, att_b1, att_W2, att_b2,
           gru_Wih, gru_Whh, gru_bih, gru_bhh):
    N, D = node_feat.shape
    E = edge.shape[0]
    DE = edge_feat.shape[1]
    C = 80
    C2 = 40
    K = 5
    EP = E // K
    eattr = edge_attributes[0]
    eft = edge_feat.T
    nattr = node_attributes[0][:, None]

    Am, Wefm, Bm, wm = (msg_W1[:D], msg_W1[D:D + DE],
                        msg_W1[D + DE:D + DE + D], msg_W1[D + DE + D:])
    Aa, Wefa, Ba, wa = (att_W1[:D], att_W1[D:D + DE],
                        att_W1[D + DE:D + DE + D], att_W1[D + DE + D:])

    P = _node_table(node_feat, eattr, nattr, (Am, Bm, wm), (Aa, Ba, wa))

    ms, dsts = [], []
    for h in range(K):
        e_h = lax.slice_in_dim(edge, h * EP, (h + 1) * EP, axis=0)
        src3d = e_h[:, 0].reshape(NW, EP // (NW * C), C)
        dst3d = e_h[:, 1].reshape(NW, EP // (NW * C), C)
        Gs, Gd = _gather_diff(P, src3d, dst3d)
        m = _edge_mlp(Gs, Gd, eft, h * EP, Wefm, msg_b1[None, :], Wefa,
                      att_b1[None, :], msg_W2, msg_b2[None, :], att_W2,
                      att_b2[None, :])
        ms.append(m)
        dsts.append(e_h[:, 1].reshape(NW, EP // (NW * C2), C2))
    parts = _scatter_add(ms, dsts, N)
    return _gru([parts], node_feat, gru_Wih, gru_Whh,
                gru_bih[None, :], gru_bhh[None, :])


# speedup vs baseline: 1.1565x; 1.0078x over previous
import functools

import jax
import jax.numpy as jnp
from jax import lax
from jax.experimental import pallas as pl
from jax.experimental.pallas import tpu as pltpu
from jax.experimental.pallas import tpu_sc as plsc

NC = 2
NS = 16
NW = NC * NS


def _node_table(state, eattr, nattr, A, B):
    N, D = state.shape
    D2 = D
    BN = 1000

    def pack16(x):
        f = x.astype(jnp.bfloat16).astype(jnp.float32)
        return jax.lax.bitcast_convert_type(f, jnp.int32)

    def body(s_ref, e_ref, na_ref, Am_ref, Bm_ref, wm_ref, Aa_ref, Ba_ref,
             wa_ref, P_ref):
        u = s_ref[...] @ Am_ref[...] + e_ref[...] @ Bm_ref[...] \
            + na_ref[...] * wm_ref[...]
        v = s_ref[...] @ Aa_ref[...] + e_ref[...] @ Ba_ref[...] \
            + na_ref[...] * wa_ref[...]
        uw = jnp.bitwise_and(jnp.right_shift(pack16(u), 16), jnp.int32(65535))
        vw = jnp.bitwise_and(pack16(v), jnp.int32(-65536))
        P_ref[...] = jnp.bitwise_or(uw, vw)

    wspec = [
        pl.BlockSpec((D, D2), lambda i: (0, 0)),
        pl.BlockSpec((D, D2), lambda i: (0, 0)),
        pl.BlockSpec((1, D2), lambda i: (0, 0)),
    ]
    return pl.pallas_call(
        body,
        grid=(N // BN,),
        in_specs=[
            pl.BlockSpec((BN, D), lambda i: (i, 0)),
            pl.BlockSpec((BN, D), lambda i: (i, 0)),
            pl.BlockSpec((BN, 1), lambda i: (i, 0)),
        ] + wspec + wspec,
        out_specs=pl.BlockSpec((BN, D2), lambda i: (i, 0)),
        out_shape=jax.ShapeDtypeStruct((N, D2), jnp.int32),
    )(state, eattr, nattr, *A, *B)


def _gather_diff(P, src3d, dst3d):
    N, D2 = P.shape
    _, NCH, C = src3d.shape
    E = NW * NCH * C
    NB = 4
    OFF = 2
    NCYC = (NCH + OFF + NB - 1) // NB

    mesh = plsc.VectorSubcoreMesh(core_axis_name="c", subcore_axis_name="s")

    @functools.partial(
        pl.kernel,
        out_type=(jax.ShapeDtypeStruct((E, D2), jnp.int32),
                  jax.ShapeDtypeStruct((E, D2), jnp.int32)),
        mesh=mesh,
        scratch_types=[
            pltpu.VMEM((NCH, C), jnp.int32),
            pltpu.VMEM((NCH, C), jnp.int32),
            pltpu.VMEM((NB, C, D2), jnp.int32),
            pltpu.VMEM((NB, C, D2), jnp.int32),
        ] + [pltpu.SemaphoreType.DMA] * (2 * NB),
    )
    def gather_k(P_hbm, src_hbm, dst_hbm, outs_hbm, outd_hbm, sbuf, dbuf,
                 bufA, bufB, *sems):
        semg = sems[:NB]
        semw = sems[NB:]
        wid = lax.axis_index("s") * NC + lax.axis_index("c")
        crow0 = wid * NCH

        pltpu.sync_copy(src_hbm.at[wid], sbuf)
        pltpu.sync_copy(dst_hbm.at[wid], dbuf)

        def drain_wb(b):
            pltpu.make_async_copy(bufA.at[b], outs_hbm.at[pl.ds(0, C)],
                                  semw[b]).wait()
            pltpu.make_async_copy(bufB.at[b], outd_hbm.at[pl.ds(0, C)],
                                  semw[b]).wait()

        def cycle(g, _):
            for b in range(NB):
                j = g * NB + b
                bw = (b - OFF) % NB
                jw = g * NB + b - OFF

                @pl.when(j < NCH)
                def _():
                    @pl.when(j >= NB)
                    def _():
                        drain_wb(b)
                    pltpu.async_copy(P_hbm.at[sbuf.at[j]], bufA.at[b], semg[b])
                    pltpu.async_copy(P_hbm.at[dbuf.at[j]], bufB.at[b], semg[b])

                @pl.when(jnp.logical_and(jw >= 0, jw < NCH))
                def _():
                    pltpu.make_async_copy(P_hbm.at[sbuf.at[bw]], bufA.at[bw],
                                          semg[bw]).wait()
                    pltpu.make_async_copy(P_hbm.at[dbuf.at[bw]], bufB.at[bw],
                                          semg[bw]).wait()
                    rows = pl.ds((crow0 + jw) * C, C)
                    pltpu.async_copy(bufA.at[bw], outs_hbm.at[rows], semw[bw])
                    pltpu.async_copy(bufB.at[bw], outd_hbm.at[rows], semw[bw])
            return 0

        lax.fori_loop(0, NCYC, cycle, 0)
        for b in range(NB):
            drain_wb(b)

    return gather_k(P, src3d, dst3d)


def _edge_mlp(Gs, Gd, eft, off, Wefm, b1m, Wefa, b1a, W2m, b2m, W2a, b2a):
    E, D = Gs.shape
    DE = eft.shape[0]
    BE = 3200
    bf = jnp.bfloat16
    cdim = (((0,), (0,)), ((), ()))

    def unpack(g):
        lo = jax.lax.bitcast_convert_type(jnp.left_shift(g, 16), jnp.float32)
        hi = jax.lax.bitcast_convert_type(
            jnp.bitwise_and(g, jnp.int32(-65536)), jnp.float32)
        return lo, hi

    def body(Gs_ref, Gd_ref, ef_ref, Wefm_ref, b1m_ref, Wefa_ref, b1a_ref,
             W2m_ref, b2m_ref, W2a_ref, b2a_ref, o_ref):
        sm, sa = unpack(Gs_ref[...])
        dm, da = unpack(Gd_ref[...])
        gm = sm - dm
        ga = sa - da
        efv = ef_ref[...]
        em = jax.lax.dot_general(efv, Wefm_ref[...], cdim,
                                 preferred_element_type=jnp.float32)
        ea = jax.lax.dot_general(efv, Wefa_ref[...], cdim,
                                 preferred_element_type=jnp.float32)
        hm = jnp.maximum(gm + em + b1m_ref[...], 0.0).astype(bf)
        ha = jnp.maximum(ga + ea + b1a_ref[...], 0.0).astype(bf)
        msg = jax.lax.dot(hm, W2m_ref[...].astype(bf),
                          preferred_element_type=jnp.float32) + b2m_ref[...]
        att = jax.lax.dot(ha, W2a_ref[...].astype(bf),
                          preferred_element_type=jnp.float32) + b2a_ref[...]
        o_ref[...] = msg * jax.nn.sigmoid(att)

    ob = off // BE
    wspec = [
        pl.BlockSpec((DE, D), lambda i: (0, 0)),
        pl.BlockSpec((1, D), lambda i: (0, 0)),
    ]
    return pl.pallas_call(
        body,
        grid=(E // BE,),
        in_specs=[
            pl.BlockSpec((BE, D), lambda i: (i, 0)),
            pl.BlockSpec((BE, D), lambda i: (i, 0)),
            pl.BlockSpec((DE, BE), lambda i: (0, i + ob)),
        ] + wspec + wspec + [
            pl.BlockSpec((D, D), lambda i: (0, 0)),
            pl.BlockSpec((1, D), lambda i: (0, 0)),
            pl.BlockSpec((D, D), lambda i: (0, 0)),
            pl.BlockSpec((1, D), lambda i: (0, 0)),
        ],
        out_specs=pl.BlockSpec((BE, D), lambda i: (i, 0)),
        out_shape=jax.ShapeDtypeStruct((E, D), jnp.float32),
    )(Gs, Gd, eft, Wefm, b1m, Wefa, b1a, W2m, b2m, W2a, b2a)


def _scatter_add(ms, dsts, N):
    K = len(ms)
    E, D = ms[0].shape
    _, NCH, C2 = dsts[0].shape
    M = 3
    NCYC = (NCH + M - 1) // M
    CZ = 16
    NZCH = N // CZ
    ZPT = (NZCH + NS - 1) // NS

    mesh = plsc.VectorSubcoreMesh(core_axis_name="c", subcore_axis_name="s")

    @functools.partial(
        pl.kernel,
        out_type=jax.ShapeDtypeStruct((NC, N, D), jnp.float32),
        mesh=mesh,
        scratch_types=[
            pltpu.VMEM_SHARED((N, D), jnp.float32),
            pltpu.VMEM((K * NCH, C2), jnp.int32),
            pltpu.VMEM((M, C2, D), jnp.float32),
            pltpu.VMEM((CZ, D), jnp.float32),
        ] + [pltpu.SemaphoreType.DMA] * (2 * M),
    )
    def scatter_k(*refs):
        m_hbms = refs[:K]
        d_hbms = refs[K:2 * K]
        out_hbm = refs[2 * K]
        acc, dbuf, mbuf, zbuf = refs[2 * K + 1:2 * K + 5]
        sems = refs[2 * K + 5:]
        semL = sems[:M]
        semS = sems[M:]
        c = lax.axis_index("c")
        s = lax.axis_index("s")
        wid = s * NC + c
        crow0 = wid * NCH

        def zrow(r, _):
            for k in range(D // 16):
                zbuf[r, pl.ds(k * 16, 16)] = jnp.zeros((16,), jnp.float32)
            return 0

        lax.fori_loop(0, CZ, zrow, 0)

        def zchunk(kk, _):
            jz = kk * NS + s

            @pl.when(jz < NZCH)
            def _():
                pltpu.sync_copy(zbuf, acc.at[pl.ds(jz * CZ, CZ)])
            return 0

        lax.fori_loop(0, ZPT, zchunk, 0)
        for h in range(K):
            pltpu.sync_copy(d_hbms[h].at[wid], dbuf.at[pl.ds(h * NCH, NCH)])
        plsc.subcore_barrier()

        for h in range(K):
            m_hbm = m_hbms[h]

            def cycle(g, _):
                for b in range(M):
                    j = g * M + b
                    bs = (b - M // 2) % M
                    js = j - M // 2

                    @pl.when(j < NCH)
                    def _():
                        @pl.when(j >= M)
                        def _():
                            pltpu.make_async_copy(
                                mbuf.at[b], acc.at[dbuf.at[0]], semS[b]).wait()
                        pltpu.async_copy(m_hbm.at[pl.ds((crow0 + j) * C2, C2)],
                                         mbuf.at[b], semL[b])

                    @pl.when(jnp.logical_and(js >= 0, js < NCH))
                    def _():
                        pltpu.make_async_copy(
                            m_hbm.at[pl.ds(0, C2)], mbuf.at[bs], semL[bs]).wait()
                        pltpu.async_copy(mbuf.at[bs],
                                         acc.at[dbuf.at[h * NCH + js]], semS[bs],
                                         add=True)
                return 0

            lax.fori_loop(0, NCYC + 1, cycle, 0)
            for b in range(M):
                pltpu.make_async_copy(mbuf.at[b], acc.at[dbuf.at[0]],
                                      semS[b]).wait()
        plsc.subcore_barrier()

        def dchunk(kk, _):
            jz = kk * NS + s

            @pl.when(jz < NZCH)
            def _():
                rows = pl.ds(jz * CZ, CZ)
                pltpu.sync_copy(acc.at[rows], zbuf)
                pltpu.sync_copy(zbuf, out_hbm.at[c, rows])
            return 0

        lax.fori_loop(0, ZPT, dchunk, 0)

    return scatter_k(*ms, *dsts)


def _gru(parts, state, Wih, Whh, bih, bhh):
    N, D = state.shape
    D3 = Wih.shape[1]
    K = len(parts)
    BN = 1000

    def body(*refs):
        p_refs = refs[:K]
        s_ref, Wih_ref, Whh_ref, bih_ref, bhh_ref, o_ref = refs[K:]
        x = p_refs[0][0] + p_refs[0][1]
        for pr in p_refs[1:]:
            x = x + pr[0] + pr[1]
        h = s_ref[...]
        gi = x @ Wih_ref[...] + bih_ref[...]
        gh = h @ Whh_ref[...] + bhh_ref[...]
        r = jax.nn.sigmoid(gi[:, :D] + gh[:, :D])
        z = jax.nn.sigmoid(gi[:, D:2 * D] + gh[:, D:2 * D])
        n = jnp.tanh(gi[:, 2 * D:] + r * gh[:, 2 * D:])
        o_ref[...] = (1.0 - z) * n + z * h

    return pl.pallas_call(
        body,
        grid=(N // BN,),
        in_specs=[pl.BlockSpec((2, BN, D), lambda i: (0, i, 0))] * K + [
            pl.BlockSpec((BN, D), lambda i: (i, 0)),
            pl.BlockSpec((D, D3), lambda i: (0, 0)),
            pl.BlockSpec((D, D3), lambda i: (0, 0)),
            pl.BlockSpec((1, D3), lambda i: (0, 0)),
            pl.BlockSpec((1, D3), lambda i: (0, 0)),
        ],
        out_specs=pl.BlockSpec((BN, D), lambda i: (i, 0)),
        out_shape=jax.ShapeDtypeStruct((N, D), jnp.float32),
    )(*parts, state, Wih, Whh, bih, bhh)


def kernel(node_feat, edge, edge_feat, node_attributes, edge_attributes,
           msg_W1, msg_b1, msg_W2, msg_b2, att_W1, att_b1, att_W2, att_b2,
           gru_Wih, gru_Whh, gru_bih, gru_bhh):
    N, D = node_feat.shape
    E = edge.shape[0]
    DE = edge_feat.shape[1]
    C = 80
    C2 = 40
    K = 5
    EP = E // K
    eattr = edge_attributes[0]
    eft = edge_feat.T
    nattr = node_attributes[0][:, None]

    Am, Wefm, Bm, wm = (msg_W1[:D], msg_W1[D:D + DE],
                        msg_W1[D + DE:D + DE + D], msg_W1[D + DE + D:])
    Aa, Wefa, Ba, wa = (att_W1[:D], att_W1[D:D + DE],
                        att_W1[D + DE:D + DE + D], att_W1[D + DE + D:])

    P = _node_table(node_feat, eattr, nattr, (Am, Bm, wm), (Aa, Ba, wa))

    ms, dsts = [], []
    for h in range(K):
        e_h = lax.slice_in_dim(edge, h * EP, (h + 1) * EP, axis=0)
        src3d = e_h[:, 0].reshape(NW, EP // (NW * C), C)
        dst3d = e_h[:, 1].reshape(NW, EP // (NW * C), C)
        Gs, Gd = _gather_diff(P, src3d, dst3d)
        m = _edge_mlp(Gs, Gd, eft, h * EP, Wefm, msg_b1[None, :], Wefa,
                      att_b1[None, :], msg_W2, msg_b2[None, :], att_W2,
                      att_b2[None, :])
        ms.append(m)
        dsts.append(e_h[:, 1].reshape(NW, EP // (NW * C2), C2))
    parts_a = _scatter_add(ms[:K - 1], dsts[:K - 1], N)
    parts_b = _scatter_add(ms[K - 1:], dsts[K - 1:], N)
    return _gru([parts_a, parts_b], node_feat, gru_Wih, gru_Whh,
                gru_bih[None, :], gru_bhh[None, :])


# speedup vs baseline: 1.1573x; 1.0007x over previous
import functools

import jax
import jax.numpy as jnp
from jax import lax
from jax.experimental import pallas as pl
from jax.experimental.pallas import tpu as pltpu
from jax.experimental.pallas import tpu_sc as plsc

NC = 2
NS = 16
NW = NC * NS


def _node_table(state, eattr, nattr, A, B):
    N, D = state.shape
    D2 = D
    BN = 1000

    def pack16(x):
        f = x.astype(jnp.bfloat16).astype(jnp.float32)
        return jax.lax.bitcast_convert_type(f, jnp.int32)

    def body(s_ref, e_ref, na_ref, Am_ref, Bm_ref, wm_ref, Aa_ref, Ba_ref,
             wa_ref, P_ref):
        u = s_ref[...] @ Am_ref[...] + e_ref[...] @ Bm_ref[...] \
            + na_ref[...] * wm_ref[...]
        v = s_ref[...] @ Aa_ref[...] + e_ref[...] @ Ba_ref[...] \
            + na_ref[...] * wa_ref[...]
        uw = jnp.bitwise_and(jnp.right_shift(pack16(u), 16), jnp.int32(65535))
        vw = jnp.bitwise_and(pack16(v), jnp.int32(-65536))
        P_ref[...] = jnp.bitwise_or(uw, vw)

    wspec = [
        pl.BlockSpec((D, D2), lambda i: (0, 0)),
        pl.BlockSpec((D, D2), lambda i: (0, 0)),
        pl.BlockSpec((1, D2), lambda i: (0, 0)),
    ]
    return pl.pallas_call(
        body,
        grid=(N // BN,),
        in_specs=[
            pl.BlockSpec((BN, D), lambda i: (i, 0)),
            pl.BlockSpec((BN, D), lambda i: (i, 0)),
            pl.BlockSpec((BN, 1), lambda i: (i, 0)),
        ] + wspec + wspec,
        out_specs=pl.BlockSpec((BN, D2), lambda i: (i, 0)),
        out_shape=jax.ShapeDtypeStruct((N, D2), jnp.int32),
    )(state, eattr, nattr, *A, *B)


def _gather_diff(P, src3d, dst3d):
    N, D2 = P.shape
    _, NCH, C = src3d.shape
    E = NW * NCH * C
    NB = 4
    OFF = 2
    NCYC = (NCH + OFF + NB - 1) // NB

    mesh = plsc.VectorSubcoreMesh(core_axis_name="c", subcore_axis_name="s")

    @functools.partial(
        pl.kernel,
        out_type=(jax.ShapeDtypeStruct((E, D2), jnp.int32),
                  jax.ShapeDtypeStruct((E, D2), jnp.int32)),
        mesh=mesh,
        scratch_types=[
            pltpu.VMEM((NCH, C), jnp.int32),
            pltpu.VMEM((NCH, C), jnp.int32),
            pltpu.VMEM((NB, C, D2), jnp.int32),
            pltpu.VMEM((NB, C, D2), jnp.int32),
        ] + [pltpu.SemaphoreType.DMA] * (2 * NB),
    )
    def gather_k(P_hbm, src_hbm, dst_hbm, outs_hbm, outd_hbm, sbuf, dbuf,
                 bufA, bufB, *sems):
        semg = sems[:NB]
        semw = sems[NB:]
        wid = lax.axis_index("s") * NC + lax.axis_index("c")
        crow0 = wid * NCH

        pltpu.sync_copy(src_hbm.at[wid], sbuf)
        pltpu.sync_copy(dst_hbm.at[wid], dbuf)

        def drain_wb(b):
            pltpu.make_async_copy(bufA.at[b], outs_hbm.at[pl.ds(0, C)],
                                  semw[b]).wait()
            pltpu.make_async_copy(bufB.at[b], outd_hbm.at[pl.ds(0, C)],
                                  semw[b]).wait()

        def cycle(g, _):
            for b in range(NB):
                j = g * NB + b
                bw = (b - OFF) % NB
                jw = g * NB + b - OFF

                @pl.when(j < NCH)
                def _():
                    @pl.when(j >= NB)
                    def _():
                        drain_wb(b)
                    pltpu.async_copy(P_hbm.at[sbuf.at[j]], bufA.at[b], semg[b])
                    pltpu.async_copy(P_hbm.at[dbuf.at[j]], bufB.at[b], semg[b])

                @pl.when(jnp.logical_and(jw >= 0, jw < NCH))
                def _():
                    pltpu.make_async_copy(P_hbm.at[sbuf.at[bw]], bufA.at[bw],
                                          semg[bw]).wait()
                    pltpu.make_async_copy(P_hbm.at[dbuf.at[bw]], bufB.at[bw],
                                          semg[bw]).wait()
                    rows = pl.ds((crow0 + jw) * C, C)
                    pltpu.async_copy(bufA.at[bw], outs_hbm.at[rows], semw[bw])
                    pltpu.async_copy(bufB.at[bw], outd_hbm.at[rows], semw[bw])
            return 0

        lax.fori_loop(0, NCYC, cycle, 0)
        for b in range(NB):
            drain_wb(b)

    return gather_k(P, src3d, dst3d)


def _edge_mlp(Gs, Gd, eft, off, Wefm, b1m, Wefa, b1a, W2m, b2m, W2a, b2a):
    E, D = Gs.shape
    DE = eft.shape[0]
    BE = 3200
    bf = jnp.bfloat16
    cdim = (((0,), (0,)), ((), ()))

    def unpack(g):
        lo = jax.lax.bitcast_convert_type(jnp.left_shift(g, 16), jnp.float32)
        hi = jax.lax.bitcast_convert_type(
            jnp.bitwise_and(g, jnp.int32(-65536)), jnp.float32)
        return lo, hi

    def body(Gs_ref, Gd_ref, ef_ref, Wefm_ref, b1m_ref, Wefa_ref, b1a_ref,
             W2m_ref, b2m_ref, W2a_ref, b2a_ref, o_ref):
        sm, sa = unpack(Gs_ref[...])
        dm, da = unpack(Gd_ref[...])
        gm = sm - dm
        ga = sa - da
        efv = ef_ref[...]
        em = jax.lax.dot_general(efv, Wefm_ref[...], cdim,
                                 preferred_element_type=jnp.float32)
        ea = jax.lax.dot_general(efv, Wefa_ref[...], cdim,
                                 preferred_element_type=jnp.float32)
        hm = jnp.maximum(gm + em + b1m_ref[...], 0.0).astype(bf)
        ha = jnp.maximum(ga + ea + b1a_ref[...], 0.0).astype(bf)
        msg = jax.lax.dot(hm, W2m_ref[...].astype(bf),
                          preferred_element_type=jnp.float32) + b2m_ref[...]
        att = jax.lax.dot(ha, W2a_ref[...].astype(bf),
                          preferred_element_type=jnp.float32) + b2a_ref[...]
        o_ref[...] = msg * jax.nn.sigmoid(att)

    ob = off // BE
    wspec = [
        pl.BlockSpec((DE, D), lambda i: (0, 0)),
        pl.BlockSpec((1, D), lambda i: (0, 0)),
    ]
    return pl.pallas_call(
        body,
        grid=(E // BE,),
        in_specs=[
            pl.BlockSpec((BE, D), lambda i: (i, 0)),
            pl.BlockSpec((BE, D), lambda i: (i, 0)),
            pl.BlockSpec((DE, BE), lambda i: (0, i + ob)),
        ] + wspec + wspec + [
            pl.BlockSpec((D, D), lambda i: (0, 0)),
            pl.BlockSpec((1, D), lambda i: (0, 0)),
            pl.BlockSpec((D, D), lambda i: (0, 0)),
            pl.BlockSpec((1, D), lambda i: (0, 0)),
        ],
        out_specs=pl.BlockSpec((BE, D), lambda i: (i, 0)),
        out_shape=jax.ShapeDtypeStruct((E, D), jnp.float32),
    )(Gs, Gd, eft, Wefm, b1m, Wefa, b1a, W2m, b2m, W2a, b2a)


def _scatter_add(ms, dsts, N):
    K = len(ms)
    E, D = ms[0].shape
    C2 = dsts[0].shape[2]
    NCHs = [d.shape[1] for d in dsts]
    NCHT = sum(NCHs)
    M = 3
    CZ = 16
    NZCH = N // CZ
    ZPT = (NZCH + NS - 1) // NS

    mesh = plsc.VectorSubcoreMesh(core_axis_name="c", subcore_axis_name="s")

    @functools.partial(
        pl.kernel,
        out_type=jax.ShapeDtypeStruct((NC, N, D), jnp.float32),
        mesh=mesh,
        scratch_types=[
            pltpu.VMEM_SHARED((N, D), jnp.float32),
            pltpu.VMEM((NCHT, C2), jnp.int32),
            pltpu.VMEM((M, C2, D), jnp.float32),
            pltpu.VMEM((CZ, D), jnp.float32),
        ] + [pltpu.SemaphoreType.DMA] * (2 * M),
    )
    def scatter_k(*refs):
        m_hbms = refs[:K]
        d_hbms = refs[K:2 * K]
        out_hbm = refs[2 * K]
        acc, dbuf, mbuf, zbuf = refs[2 * K + 1:2 * K + 5]
        sems = refs[2 * K + 5:]
        semL = sems[:M]
        semS = sems[M:]
        c = lax.axis_index("c")
        s = lax.axis_index("s")
        wid = s * NC + c

        def zrow(r, _):
            for k in range(D // 16):
                zbuf[r, pl.ds(k * 16, 16)] = jnp.zeros((16,), jnp.float32)
            return 0

        lax.fori_loop(0, CZ, zrow, 0)

        def zchunk(kk, _):
            jz = kk * NS + s

            @pl.when(jz < NZCH)
            def _():
                pltpu.sync_copy(zbuf, acc.at[pl.ds(jz * CZ, CZ)])
            return 0

        lax.fori_loop(0, ZPT, zchunk, 0)
        doff = 0
        for h in range(K):
            pltpu.sync_copy(d_hbms[h].at[wid], dbuf.at[pl.ds(doff, NCHs[h])])
            doff += NCHs[h]
        plsc.subcore_barrier()

        doff = 0
        for h in range(K):
            m_hbm = m_hbms[h]
            NCH = NCHs[h]
            crow0 = wid * NCH
            dbase = doff
            NCYC = (NCH + M - 1) // M

            def cycle(g, _):
                for b in range(M):
                    j = g * M + b
                    bs = (b - M // 2) % M
                    js = j - M // 2

                    @pl.when(j < NCH)
                    def _():
                        @pl.when(j >= M)
                        def _():
                            pltpu.make_async_copy(
                                mbuf.at[b], acc.at[dbuf.at[0]], semS[b]).wait()
                        pltpu.async_copy(m_hbm.at[pl.ds((crow0 + j) * C2, C2)],
                                         mbuf.at[b], semL[b])

                    @pl.when(jnp.logical_and(js >= 0, js < NCH))
                    def _():
                        pltpu.make_async_copy(
                            m_hbm.at[pl.ds(0, C2)], mbuf.at[bs], semL[bs]).wait()
                        pltpu.async_copy(mbuf.at[bs],
                                         acc.at[dbuf.at[dbase + js]], semS[bs],
                                         add=True)
                return 0

            lax.fori_loop(0, NCYC + 1, cycle, 0)
            for b in range(M):
                pltpu.make_async_copy(mbuf.at[b], acc.at[dbuf.at[0]],
                                      semS[b]).wait()
            doff += NCH
        plsc.subcore_barrier()

        def dchunk(kk, _):
            jz = kk * NS + s

            @pl.when(jz < NZCH)
            def _():
                rows = pl.ds(jz * CZ, CZ)
                pltpu.sync_copy(acc.at[rows], zbuf)
                pltpu.sync_copy(zbuf, out_hbm.at[c, rows])
            return 0

        lax.fori_loop(0, ZPT, dchunk, 0)

    return scatter_k(*ms, *dsts)


def _gru(parts, state, Wih, Whh, bih, bhh):
    N, D = state.shape
    D3 = Wih.shape[1]
    K = len(parts)
    BN = 1000

    def body(*refs):
        p_refs = refs[:K]
        s_ref, Wih_ref, Whh_ref, bih_ref, bhh_ref, o_ref = refs[K:]
        x = p_refs[0][0] + p_refs[0][1]
        for pr in p_refs[1:]:
            x = x + pr[0] + pr[1]
        h = s_ref[...]
        gi = x @ Wih_ref[...] + bih_ref[...]
        gh = h @ Whh_ref[...] + bhh_ref[...]
        r = jax.nn.sigmoid(gi[:, :D] + gh[:, :D])
        z = jax.nn.sigmoid(gi[:, D:2 * D] + gh[:, D:2 * D])
        n = jnp.tanh(gi[:, 2 * D:] + r * gh[:, 2 * D:])
        o_ref[...] = (1.0 - z) * n + z * h

    return pl.pallas_call(
        body,
        grid=(N // BN,),
        in_specs=[pl.BlockSpec((2, BN, D), lambda i: (0, i, 0))] * K + [
            pl.BlockSpec((BN, D), lambda i: (i, 0)),
            pl.BlockSpec((D, D3), lambda i: (0, 0)),
            pl.BlockSpec((D, D3), lambda i: (0, 0)),
            pl.BlockSpec((1, D3), lambda i: (0, 0)),
            pl.BlockSpec((1, D3), lambda i: (0, 0)),
        ],
        out_specs=pl.BlockSpec((BN, D), lambda i: (i, 0)),
        out_shape=jax.ShapeDtypeStruct((N, D), jnp.float32),
    )(*parts, state, Wih, Whh, bih, bhh)


def kernel(node_feat, edge, edge_feat, node_attributes, edge_attributes,
           msg_W1, msg_b1, msg_W2, msg_b2, att_W1, att_b1, att_W2, att_b2,
           gru_Wih, gru_Whh, gru_bih, gru_bhh):
    N, D = node_feat.shape
    E = edge.shape[0]
    DE = edge_feat.shape[1]
    C = 80
    C2 = 40
    GRP = NW * C
    groups = [10, 25, 30, 30, 30]
    eattr = edge_attributes[0]
    eft = edge_feat.T
    nattr = node_attributes[0][:, None]

    Am, Wefm, Bm, wm = (msg_W1[:D], msg_W1[D:D + DE],
                        msg_W1[D + DE:D + DE + D], msg_W1[D + DE + D:])
    Aa, Wefa, Ba, wa = (att_W1[:D], att_W1[D:D + DE],
                        att_W1[D + DE:D + DE + D], att_W1[D + DE + D:])

    P = _node_table(node_feat, eattr, nattr, (Am, Bm, wm), (Aa, Ba, wa))

    ms, dsts = [], []
    e0 = 0
    for g in groups:
        ep = g * GRP
        e_h = lax.slice_in_dim(edge, e0, e0 + ep, axis=0)
        src3d = e_h[:, 0].reshape(NW, g, C)
        dst3d = e_h[:, 1].reshape(NW, g, C)
        Gs, Gd = _gather_diff(P, src3d, dst3d)
        m = _edge_mlp(Gs, Gd, eft, e0, Wefm, msg_b1[None, :], Wefa,
                      att_b1[None, :], msg_W2, msg_b2[None, :], att_W2,
                      att_b2[None, :])
        ms.append(m)
        dsts.append(e_h[:, 1].reshape(NW, 2 * g, C2))
        e0 += ep
    parts_a = _scatter_add(ms[:3], dsts[:3], N)
    parts_b = _scatter_add(ms[3:], dsts[3:], N)
    return _gru([parts_a, parts_b], node_feat, gru_Wih, gru_Whh,
                gru_bih[None, :], gru_bhh[None, :])
